# baseline (speedup 1.0000x reference)
import functools

import jax
import jax.numpy as jnp
from jax import lax
from jax.experimental import pallas as pl
from jax.experimental.pallas import tpu as pltpu
from jax.experimental.pallas import tpu_sc as plsc

_VOCAB = 100000
_EMBED = 16
_CTX = 20
_BATCH = 1024

_NC = 2
_NS = 16
_NW = _NC * _NS
_ROWS_W = _BATCH // _NW
_IDX_W = _ROWS_W * _CTX
_CHUNK = 128
_NCHUNK = _IDX_W // _CHUNK

_PACK = 8
_PROWS = _VOCAB // _PACK
_SLAB = 3200
_TAIL = 768
_ALIGNED_WORDS = _SLAB * (_NW - 1) + _TAIL
_NTAIL = _VOCAB - _ALIGNED_WORDS
_TAIL_ROW0 = _ALIGNED_WORDS // _PACK
_SLAB_PR = _SLAB // _PACK
_TAIL_PR = _TAIL // _PACK


@functools.lru_cache(maxsize=1)
def _build_pack_sc():
    mesh = plsc.VectorSubcoreMesh(core_axis_name="c", subcore_axis_name="s")

    @functools.partial(
        pl.kernel,
        mesh=mesh,
        out_type=jax.ShapeDtypeStruct((_PROWS, _PACK * _EMBED), jnp.float32),
        scratch_types=[
            pltpu.VMEM((_EMBED, _SLAB), jnp.float32),
            pltpu.VMEM((_SLAB_PR, _PACK * _EMBED), jnp.float32),
        ],
        compiler_params=pltpu.CompilerParams(needs_layout_passes=False),
    )
    def pack_sc(tt_hbm, out_hbm, slab_v, out_v):
        wid = lax.axis_index("s") * _NC + lax.axis_index("c")
        base = wid * _SLAB
        lanes = lax.iota(jnp.int32, _EMBED)

        @pl.when(wid != _NW - 1)
        def _():
            for h in range(_EMBED // 8):
                pltpu.sync_copy(
                    tt_hbm.at[pl.ds(8 * h, 8), pl.ds(base, _SLAB)],
                    slab_v.at[pl.ds(8 * h, 8), :],
                )

        @pl.when(wid == _NW - 1)
        def _():
            for h in range(_EMBED // 8):
                pltpu.sync_copy(
                    tt_hbm.at[pl.ds(8 * h, 8), pl.ds(base, _TAIL)],
                    slab_v.at[pl.ds(8 * h, 8), pl.ds(0, _TAIL)],
                )

        def prow_body(p, _):
            for s in range(_PACK):
                j = p * _PACK + s
                vec = plsc.load_gather(slab_v, [lanes, jnp.zeros(
                    (_EMBED,), jnp.int32) + j])
                out_v[p, pl.ds(s * _EMBED, _EMBED)] = vec
            return 0

        lax.fori_loop(0, _SLAB_PR, prow_body, 0)

        @pl.when(wid != _NW - 1)
        def _():
            pltpu.sync_copy(out_v, out_hbm.at[pl.ds(wid * _SLAB_PR, _SLAB_PR)])

        @pl.when(wid == _NW - 1)
        def _():
            pltpu.sync_copy(
                out_v.at[pl.ds(0, _TAIL_PR)],
                out_hbm.at[pl.ds(wid * _SLAB_PR, _TAIL_PR)],
            )

    return pack_sc


@functools.lru_cache(maxsize=1)
def _build_pool_sc():
    mesh = plsc.VectorSubcoreMesh(core_axis_name="c", subcore_axis_name="s")

    @functools.partial(
        pl.kernel,
        mesh=mesh,
        out_type=jax.ShapeDtypeStruct((_BATCH, _EMBED), jnp.float32),
        scratch_types=[
            pltpu.VMEM((_IDX_W,), jnp.int32),
            pltpu.VMEM((_IDX_W, _PACK * _EMBED), jnp.float32),
            pltpu.VMEM((_ROWS_W, _EMBED), jnp.float32),
            pltpu.VMEM((_IDX_W,), jnp.int32),
            pltpu.VMEM((_NTAIL * _EMBED,), jnp.float32),
            pltpu.SemaphoreType.DMA,
        ],
        compiler_params=pltpu.CompilerParams(needs_layout_passes=False),
    )
    def pool_sc(ridx_hbm, soff_hbm, table_hbm, tail_hbm, out_hbm,
                ridx_v, rows_v, out_v, soff_v, tail_v, sem):
        wid = lax.axis_index("s") * _NC + lax.axis_index("c")
        base = wid * _IDX_W
        pltpu.sync_copy(ridx_hbm.at[pl.ds(base, _IDX_W)], ridx_v)
        pltpu.sync_copy(soff_hbm.at[pl.ds(base, _IDX_W)], soff_v)
        pltpu.sync_copy(tail_hbm, tail_v)
        copies = []
        for j in range(_NCHUNK):
            copies.append(
                pltpu.async_copy(
                    table_hbm.at[ridx_v.at[pl.ds(j * _CHUNK, _CHUNK)]],
                    rows_v.at[pl.ds(j * _CHUNK, _CHUNK)],
                    sem,
                )
            )
        for c in copies:
            c.wait()
        inv = jnp.float32(1.0 / _CTX)
        lanes = lax.iota(jnp.int32, _EMBED)
        for half in range(_ROWS_W // _EMBED):
            row_vec = half * _EMBED + lanes
            acc = [jnp.zeros((_EMBED,), jnp.float32) for _ in range(_EMBED)]
            for t in range(_CTX):
                slot_vec = row_vec * _CTX + t
                soff = plsc.load_gather(soff_v, [slot_vec])
                ridx = plsc.load_gather(ridx_v, [slot_vec])
                tail_mask = ridx >= _TAIL_ROW0
                tfl = jnp.clip((ridx - _TAIL_ROW0) * (_PACK * _EMBED) + soff,
                               0, _NTAIL * _EMBED - _EMBED)
                for d in range(_EMBED):
                    vals = plsc.load_gather(rows_v, [slot_vec, soff + d])
                    tvals = plsc.load_gather(tail_v, [tfl + d])
                    acc[d] = acc[d] + jnp.where(tail_mask, tvals, vals)
            for d in range(_EMBED):
                plsc.store_scatter(
                    out_v, [row_vec, jnp.full((_EMBED,), d, jnp.int32)],
                    acc[d] * inv,
                )
        pltpu.sync_copy(out_v, out_hbm.at[pl.ds(wid * _ROWS_W, _ROWS_W)])

    return pool_sc


_VB = 4096
_GRID = (_VOCAB + _VB - 1) // _VB


def _proj_body(wt_ref, emb_ref, b_ref, out_ref):
    out_ref[...] = (
        lax.dot_general(
            wt_ref[...],
            emb_ref[...],
            dimension_numbers=(((0,), (1,)), ((), ())),
            preferred_element_type=jnp.float32,
        )
        + b_ref[...].T
    )


def _project_t(pooled, Wt, b2):
    return pl.pallas_call(
        _proj_body,
        grid=(_GRID,),
        in_specs=[
            pl.BlockSpec((_EMBED, _VB), lambda i: (0, i)),
            pl.BlockSpec((_BATCH, _EMBED), lambda i: (0, 0)),
            pl.BlockSpec((1, _VB), lambda i: (0, i)),
        ],
        out_specs=pl.BlockSpec((_VB, _BATCH), lambda i: (i, 0)),
        out_shape=jax.ShapeDtypeStruct((_VOCAB, _BATCH), jnp.float32),
    )(Wt, pooled, b2)


def kernel(context_words, target_word, emb_table, W, b):
    del target_word
    idx = context_words.astype(jnp.int32).reshape(_BATCH * _CTX)
    ridx = idx >> 3
    soff = (idx & 7) * _EMBED
    packed = _build_pack_sc()(emb_table.T)
    tail = emb_table[_ALIGNED_WORDS:, :].reshape(_NTAIL * _EMBED)
    pooled = _build_pool_sc()(ridx, soff, packed, tail)
    logits_t = _project_t(pooled, W.T, b.reshape(1, _VOCAB))
    return logits_t.T


# speedup vs baseline: 1.1897x; 1.1897x over previous
import functools

import jax
import jax.numpy as jnp
from jax import lax
from jax.experimental import pallas as pl
from jax.experimental.pallas import tpu as pltpu
from jax.experimental.pallas import tpu_sc as plsc

_VOCAB = 100000
_EMBED = 16
_CTX = 20
_BATCH = 1024

_NC = 2
_NS = 16
_NW = _NC * _NS
_ROWS_W = _BATCH // _NW
_IDX_W = _ROWS_W * _CTX
_CHUNK = 128
_NCHUNK = _IDX_W // _CHUNK

@functools.lru_cache(maxsize=1)
def _build_pool_sc():
    mesh = plsc.VectorSubcoreMesh(core_axis_name="c", subcore_axis_name="s")

    @functools.partial(
        pl.kernel,
        mesh=mesh,
        out_type=jax.ShapeDtypeStruct((_BATCH, _EMBED), jnp.float32),
        scratch_types=[
            pltpu.VMEM((_IDX_W,), jnp.int32),
            pltpu.VMEM((_IDX_W, _EMBED), jnp.float32),
            pltpu.VMEM((_ROWS_W, _EMBED), jnp.float32),
            pltpu.SemaphoreType.DMA,
        ],
        compiler_params=pltpu.CompilerParams(use_tc_tiling_on_sc=False),
    )
    def pool_sc(idx_hbm, table_hbm, out_hbm, idx_v, rows_v, out_v, sem):
        wid = lax.axis_index("s") * _NC + lax.axis_index("c")
        pltpu.sync_copy(idx_hbm.at[pl.ds(wid * _IDX_W, _IDX_W)], idx_v)
        copies = []
        for j in range(_NCHUNK):
            copies.append(
                pltpu.async_copy(
                    table_hbm.at[idx_v.at[pl.ds(j * _CHUNK, _CHUNK)]],
                    rows_v.at[pl.ds(j * _CHUNK, _CHUNK)],
                    sem,
                )
            )
        for c in copies:
            c.wait()
        inv = jnp.float32(1.0 / _CTX)
        for r in range(_ROWS_W):
            acc = rows_v[r * _CTX]
            for t in range(1, _CTX):
                acc = acc + rows_v[r * _CTX + t]
            out_v[r] = acc * inv
        pltpu.sync_copy(out_v, out_hbm.at[pl.ds(wid * _ROWS_W, _ROWS_W)])

    return pool_sc


_PCOL = 8192
_PGRID = (_VOCAB + _PCOL - 1) // _PCOL


def _pack_body(wt_ref, out_ref):
    y = wt_ref[...].T.reshape(_PCOL // 8, 8, _EMBED)
    for s in range(8):
        out_ref[:, s * _EMBED:(s + 1) * _EMBED] = y[:, s, :]


def _pack_table(Wt_view):
    return pl.pallas_call(
        _pack_body,
        grid=(_PGRID,),
        in_specs=[pl.BlockSpec((_EMBED, _PCOL), lambda i: (0, i))],
        out_specs=pl.BlockSpec((_PCOL * _EMBED // 128, 128), lambda i: (i, 0)),
        out_shape=jax.ShapeDtypeStruct((_VOCAB * _EMBED // 128, 128),
                                       jnp.float32),
    )(Wt_view)


_VB = 4096
_GRID = (_VOCAB + _VB - 1) // _VB


def _proj_body(wt_ref, emb_ref, b_ref, out_ref):
    out_ref[...] = (
        lax.dot_general(
            wt_ref[...],
            emb_ref[...],
            dimension_numbers=(((0,), (1,)), ((), ())),
            preferred_element_type=jnp.float32,
        )
        + b_ref[...].T
    )


def _project_t(pooled, Wt, b2):
    return pl.pallas_call(
        _proj_body,
        grid=(_GRID,),
        in_specs=[
            pl.BlockSpec((_EMBED, _VB), lambda i: (0, i)),
            pl.BlockSpec((_BATCH, _EMBED), lambda i: (0, 0)),
            pl.BlockSpec((1, _VB), lambda i: (0, i)),
        ],
        out_specs=pl.BlockSpec((_VB, _BATCH), lambda i: (i, 0)),
        out_shape=jax.ShapeDtypeStruct((_VOCAB, _BATCH), jnp.float32),
    )(Wt, pooled, b2)


def kernel(context_words, target_word, emb_table, W, b):
    del target_word
    idx = context_words.astype(jnp.int32).reshape(_BATCH * _CTX)
    packed = _pack_table(emb_table.T)
    pooled = _build_pool_sc()(idx, packed.reshape(_VOCAB, _EMBED))
    logits_t = _project_t(pooled, W.T, b.reshape(1, _VOCAB))
    return logits_t.T
